# baseline (speedup 1.0000x reference)
import functools

import jax
import jax.numpy as jnp
from jax import lax
from jax.experimental import pallas as pl
from jax.experimental.pallas import tpu as pltpu
from jax.experimental.pallas import tpu_sc as plsc

NROW = 16384
NCOL = 50
D = 32
NC, NS = 2, 16
NW = NC * NS
RPW = NROW // NW
R = 32
NCHUNK = RPW // R
C = R * NCOL

_mesh = plsc.VectorSubcoreMesh(core_axis_name="c", subcore_axis_name="s")


@functools.partial(
    pl.kernel,
    out_type=jax.ShapeDtypeStruct((NCOL, 4, 128, 8, 128), jnp.float32),
    mesh=_mesh,
    scratch_types=[
        pltpu.VMEM((R, NCOL), jnp.int32),
        pltpu.VMEM((C,), jnp.int32),
        pltpu.VMEM((C, D), jnp.float32),
        pltpu.VMEM((NCOL, 4, 8, R), jnp.float32),
        pltpu.SemaphoreType.DMA,
        pltpu.SemaphoreType.DMA,
    ],
    compiler_params=pltpu.CompilerParams(
        use_tc_tiling_on_sc=False, needs_layout_passes=False),
)
def _gather_kernel(x_hbm, table_hbm, out_hbm, xchunk_v, idx_v, rows_v,
                   slab_v, sem, sem_wb):
    wid = lax.axis_index("s") * NC + lax.axis_index("c")
    base_row = wid * RPW
    lanes = lax.iota(jnp.int32, 16)

    def chunk_body(c, _):
        r0 = base_row + c * R
        it = lax.div(r0, 128)
        il0 = lax.rem(r0, 128)
        pltpu.sync_copy(x_hbm.at[pl.ds(r0, R), :], xchunk_v)

        def idx_step(t, carry):
            k = t * 16 + lanes
            l = jnp.bitwise_and(k, R - 1)
            j = jnp.right_shift(k, 5)
            vals = plsc.load_gather(xchunk_v, [l, j])
            idx_v[pl.ds(t * 16, 16)] = vals
            return carry

        lax.fori_loop(0, C // 16, idx_step, 0)

        pltpu.async_copy(table_hbm.at[idx_v], rows_v, sem).wait()

        def tr_step(t, carry):
            half = jnp.bitwise_and(t, 1)
            ds = jnp.bitwise_and(jnp.right_shift(t, 1), 7)
            dt = jnp.bitwise_and(jnp.right_shift(t, 4), 3)
            j = jnp.right_shift(t, 6)
            il_ = half * 16
            row = j * R + il_ + lanes
            col = jnp.full((16,), jnp.int32(0)) + (8 * dt + ds)
            vals = plsc.load_gather(rows_v, [row, col])
            slab_v[j, dt, ds, pl.ds(il_, 16)] = vals
            return carry

        lax.fori_loop(0, NCOL * 4 * 8 * (R // 16), tr_step, 0)

        wb = [
            pltpu.async_copy(
                slab_v.at[j, dt],
                out_hbm.at[j, dt, it, :, pl.ds(il0, R)], sem_wb)
            for j in range(NCOL)
            for dt in range(4)
        ]
        for d_ in wb:
            d_.wait()
        return _

    lax.fori_loop(0, NCHUNK, chunk_body, 0)


def kernel(x, embedding):
    o5 = _gather_kernel(x, embedding)
    return o5.transpose(2, 4, 0, 1, 3).reshape(NROW, NCOL, D)


# speedup vs baseline: 1.6651x; 1.6651x over previous
import functools

import jax
import jax.numpy as jnp
from jax import lax
from jax.experimental import pallas as pl
from jax.experimental.pallas import tpu as pltpu
from jax.experimental.pallas import tpu_sc as plsc

NROW = 16384
NCOL = 50
D = 32
NC, NS = 2, 16
NW = NC * NS
RPW = NROW // NW
R = 32
NCHUNK = RPW // R
C = R * NCOL

_mesh = plsc.VectorSubcoreMesh(core_axis_name="c", subcore_axis_name="s")


@functools.partial(
    pl.kernel,
    out_type=jax.ShapeDtypeStruct((NROW, 56, 128), jnp.float32),
    mesh=_mesh,
    scratch_types=[
        pltpu.VMEM((R, NCOL), jnp.int32),
        pltpu.VMEM((R, NCOL), jnp.int32),
        pltpu.VMEM((C,), jnp.int32),
        pltpu.VMEM((C,), jnp.int32),
        pltpu.VMEM((2, C, D), jnp.float32),
        pltpu.SemaphoreType.DMA,
        pltpu.SemaphoreType.DMA,
        pltpu.SemaphoreType.DMA,
        pltpu.SemaphoreType.DMA,
    ],
    compiler_params=pltpu.CompilerParams(
        use_tc_tiling_on_sc=False, needs_layout_passes=False),
)
def _gather_kernel(x_hbm, table_hbm, out_hbm, xc0, xc1, ix0, ix1, rows_v,
                   sg0, sg1, sw0, sw1):
    xc = [xc0, xc1]
    ix = [ix0, ix1]
    sg = [sg0, sg1]
    sw = [sw0, sw1]
    wid = lax.axis_index("s") * NC + lax.axis_index("c")
    base_row = wid * RPW
    lanes = lax.iota(jnp.int32, 16)

    wb_prev = []
    for c in range(NCHUNK):
        b = c & 1
        r0 = base_row + c * R
        pltpu.sync_copy(x_hbm.at[pl.ds(r0, R), :], xc[b])

        def idx_step(t, carry, b=b):
            k = t * 16 + lanes
            l = jnp.bitwise_and(k, R - 1)
            j = jnp.right_shift(k, 5)
            vals = plsc.load_gather(xc[b], [l, j])
            ix[b][pl.ds(t * 16, 16)] = vals
            return carry

        lax.fori_loop(0, C // 16, idx_step, 0)

        g = pltpu.async_copy(table_hbm.at[ix[b]], rows_v.at[b], sg[b])
        for d_ in wb_prev:
            d_.wait()
        g.wait()
        wb_prev = [
            pltpu.async_copy(
                rows_v.at[b, pl.ds(j * R, R), :],
                out_hbm.at[pl.ds(r0, R), j, pl.ds(0, D)], sw[b])
            for j in range(NCOL)
        ]
    for d_ in wb_prev:
        d_.wait()


def kernel(x, embedding):
    out_padded = _gather_kernel(x, embedding)
    return out_padded[:, :NCOL, :D]


# speedup vs baseline: 1.6794x; 1.0086x over previous
import functools

import jax
import jax.numpy as jnp
from jax import lax
from jax.experimental import pallas as pl
from jax.experimental.pallas import tpu as pltpu
from jax.experimental.pallas import tpu_sc as plsc

NROW = 16384
NCOL = 50
D = 32
NC, NS = 2, 16
NW = NC * NS
RPW = NROW // NW
R = 64
NCHUNK = RPW // R
C = R * NCOL

_mesh = plsc.VectorSubcoreMesh(core_axis_name="c", subcore_axis_name="s")


@functools.partial(
    pl.kernel,
    out_type=jax.ShapeDtypeStruct((NROW, 56, 128), jnp.float32),
    mesh=_mesh,
    scratch_types=[
        pltpu.VMEM((R, NCOL), jnp.int32),
        pltpu.VMEM((C,), jnp.int32),
        pltpu.VMEM((C, D), jnp.float32),
        pltpu.SemaphoreType.DMA,
        pltpu.SemaphoreType.DMA,
    ],
    compiler_params=pltpu.CompilerParams(
        use_tc_tiling_on_sc=False, needs_layout_passes=False),
)
def _gather_kernel(x_hbm, table_hbm, out_hbm, xchunk_v, idx_v, rows_v, sem,
                   sem_wb):
    wid = lax.axis_index("s") * NC + lax.axis_index("c")
    base_row = wid * RPW
    lanes = lax.iota(jnp.int32, 16)

    for c in range(NCHUNK):
        r0 = base_row + c * R
        pltpu.sync_copy(x_hbm.at[pl.ds(r0, R), :], xchunk_v)

        def transpose_step(t, _):
            k = t * 16 + lanes
            r = jnp.bitwise_and(k, R - 1)
            j = jnp.right_shift(k, 6)
            vals = plsc.load_gather(xchunk_v, [r, j])
            idx_v[pl.ds(t * 16, 16)] = vals
            return _

        lax.fori_loop(0, C // 16, transpose_step, 0)

        pltpu.async_copy(table_hbm.at[idx_v], rows_v, sem).wait()

        wb = [
            pltpu.async_copy(
                rows_v.at[pl.ds(j * R, R), :],
                out_hbm.at[pl.ds(r0, R), j, pl.ds(0, D)], sem_wb)
            for j in range(NCOL)
        ]
        for d in wb:
            d.wait()


def kernel(x, embedding):
    out_padded = _gather_kernel(x, embedding)
    return out_padded[:, :NCOL, :D]
